# baseline (device time: 51864 ns/iter reference)
import jax
import jax.numpy as jnp
from jax import lax
from jax.experimental import pallas as pl
from jax.experimental.pallas import tpu as pltpu

N_DEV = 4
M = 1024
D = 1024
CHUNK = M // N_DEV


def kernel(x, Wg, Wu, Wd):
    def body(x_ref, wg_ref, wu_ref, wd_ref, out_ref,
             xb_buf, wgb_buf, wub_buf, wdb_buf,
             part_buf, own_buf, red_buf, rs_buf, ag_buf,
             send_sems, rs_sems, ag_sems):
        my = lax.axis_index("i")
        right = lax.rem(my + 1, N_DEV)
        left = lax.rem(my + N_DEV - 1, N_DEV)
        diag = lax.rem(my + 2, N_DEV)

        barrier_sem = pltpu.get_barrier_semaphore()
        for nbr in (right, left, diag):
            pl.semaphore_signal(
                barrier_sem, inc=1,
                device_id=(nbr,), device_id_type=pl.DeviceIdType.MESH,
            )
        pl.semaphore_wait(barrier_sem, 3)

        xb_buf[...] = x_ref[...].astype(jnp.bfloat16)
        wgb_buf[...] = wg_ref[...].astype(jnp.bfloat16)
        wub_buf[...] = wu_ref[...].astype(jnp.bfloat16)
        wdb_buf[...] = wd_ref[...].astype(jnp.bfloat16)

        def partial_chunk(c):
            xs = xb_buf[pl.ds(c * CHUNK, CHUNK), :]
            g = jnp.dot(xs, wgb_buf[...], preferred_element_type=jnp.float32)
            u = jnp.dot(xs, wub_buf[...], preferred_element_type=jnp.float32)
            h = (g * (u * lax.logistic(u))).astype(jnp.bfloat16)
            return jnp.dot(h, wdb_buf[...], preferred_element_type=jnp.float32)

        def rs_send(target, src_slot, slot):
            return pltpu.make_async_remote_copy(
                src_ref=part_buf.at[src_slot],
                dst_ref=rs_buf.at[slot],
                send_sem=send_sems.at[slot],
                recv_sem=rs_sems.at[slot],
                device_id=(target,),
                device_id_type=pl.DeviceIdType.MESH,
            )

        part_buf[0] = partial_chunk(diag).astype(jnp.bfloat16)
        rs_diag = rs_send(diag, 0, 2)
        rs_diag.start()
        part_buf[1] = partial_chunk(right).astype(jnp.bfloat16)
        rs_right = rs_send(right, 1, 0)
        rs_right.start()
        part_buf[2] = partial_chunk(left).astype(jnp.bfloat16)
        rs_left = rs_send(left, 2, 1)
        rs_left.start()
        own_buf[...] = partial_chunk(my)

        rs_right.wait_recv()
        rs_left.wait_recv()
        rs_diag.wait_recv()
        red = (own_buf[...] + rs_buf[0].astype(jnp.float32)) + (
            rs_buf[1].astype(jnp.float32) + rs_buf[2].astype(jnp.float32))
        red_buf[...] = red.astype(jnp.bfloat16)

        def ag_send(target, slot):
            return pltpu.make_async_remote_copy(
                src_ref=red_buf,
                dst_ref=ag_buf.at[slot],
                send_sem=send_sems.at[3 + slot],
                recv_sem=ag_sems.at[slot],
                device_id=(target,),
                device_id_type=pl.DeviceIdType.MESH,
            )

        ag_diag = ag_send(diag, 2)
        ag_diag.start()
        ag_right = ag_send(right, 0)
        ag_right.start()
        ag_left = ag_send(left, 1)
        ag_left.start()

        out_ref[pl.ds(my * CHUNK, CHUNK), :] = red

        ag_right.wait_recv()
        out_ref[pl.ds(left * CHUNK, CHUNK), :] = ag_buf[0].astype(jnp.float32)
        ag_left.wait_recv()
        out_ref[pl.ds(right * CHUNK, CHUNK), :] = ag_buf[1].astype(jnp.float32)
        ag_diag.wait_recv()
        out_ref[pl.ds(diag * CHUNK, CHUNK), :] = ag_buf[2].astype(jnp.float32)

        rs_right.wait_send()
        rs_left.wait_send()
        rs_diag.wait_send()
        ag_right.wait_send()
        ag_left.wait_send()
        ag_diag.wait_send()

    return pl.pallas_call(
        body,
        out_shape=jax.ShapeDtypeStruct((M, D), jnp.float32),
        in_specs=[
            pl.BlockSpec(memory_space=pltpu.VMEM),
            pl.BlockSpec(memory_space=pltpu.VMEM),
            pl.BlockSpec(memory_space=pltpu.VMEM),
            pl.BlockSpec(memory_space=pltpu.VMEM),
        ],
        out_specs=pl.BlockSpec(memory_space=pltpu.VMEM),
        scratch_shapes=[
            pltpu.VMEM((M, 1024), jnp.bfloat16),
            pltpu.VMEM((1024, 2048), jnp.bfloat16),
            pltpu.VMEM((1024, 2048), jnp.bfloat16),
            pltpu.VMEM((2048, 1024), jnp.bfloat16),
            pltpu.VMEM((3, CHUNK, D), jnp.bfloat16),
            pltpu.VMEM((CHUNK, D), jnp.float32),
            pltpu.VMEM((CHUNK, D), jnp.bfloat16),
            pltpu.VMEM((3, CHUNK, D), jnp.bfloat16),
            pltpu.VMEM((3, CHUNK, D), jnp.bfloat16),
            pltpu.SemaphoreType.DMA((6,)),
            pltpu.SemaphoreType.DMA((3,)),
            pltpu.SemaphoreType.DMA((3,)),
        ],
        compiler_params=pltpu.CompilerParams(
            collective_id=0,
            vmem_limit_bytes=128 * 1024 * 1024,
        ),
    )(x, Wg, Wu, Wd)


# device time: 50895 ns/iter; 1.0190x vs baseline; 1.0190x over previous
import jax
import jax.numpy as jnp
from jax import lax
from jax.experimental import pallas as pl
from jax.experimental.pallas import tpu as pltpu

N_DEV = 4
M = 1024
D = 1024
CHUNK = M // N_DEV
HALF = CHUNK // 2
N_WAVES = 2


def kernel(x, Wg, Wu, Wd):
    def body(x_ref, wg_ref, wu_ref, wd_ref, out_ref,
             xb_buf, wgb_buf, wub_buf, wdb_buf,
             part_buf, own_buf, red_buf, rs_buf, ag_buf,
             send_sems, rs_sems, ag_sems):
        my = lax.axis_index("i")
        right = lax.rem(my + 1, N_DEV)
        left = lax.rem(my + N_DEV - 1, N_DEV)
        diag = lax.rem(my + 2, N_DEV)

        barrier_sem = pltpu.get_barrier_semaphore()
        for nbr in (right, left, diag):
            pl.semaphore_signal(
                barrier_sem, inc=1,
                device_id=(nbr,), device_id_type=pl.DeviceIdType.MESH,
            )
        pl.semaphore_wait(barrier_sem, 3)

        xb_buf[...] = x_ref[...].astype(jnp.bfloat16)
        wgb_buf[...] = wg_ref[...].astype(jnp.bfloat16)
        wub_buf[...] = wu_ref[...].astype(jnp.bfloat16)
        wdb_buf[...] = wd_ref[...].astype(jnp.bfloat16)

        def partial_half(c, w):
            xs = xb_buf[pl.ds(c * CHUNK + w * HALF, HALF), :]
            g = jnp.dot(xs, wgb_buf[...], preferred_element_type=jnp.float32)
            u = jnp.dot(xs, wub_buf[...], preferred_element_type=jnp.float32)
            h = (g * (u * lax.logistic(u))).astype(jnp.bfloat16)
            return jnp.dot(h, wdb_buf[...], preferred_element_type=jnp.float32)

        def rs_send(target, w, slot):
            return pltpu.make_async_remote_copy(
                src_ref=part_buf.at[w, slot],
                dst_ref=rs_buf.at[w, slot],
                send_sem=send_sems.at[w * 6 + slot],
                recv_sem=rs_sems.at[w * 3 + slot],
                device_id=(target,),
                device_id_type=pl.DeviceIdType.MESH,
            )

        def ag_send(target, w, slot):
            return pltpu.make_async_remote_copy(
                src_ref=red_buf.at[w],
                dst_ref=ag_buf.at[w, slot],
                send_sem=send_sems.at[w * 6 + 3 + slot],
                recv_sem=ag_sems.at[w * 3 + slot],
                device_id=(target,),
                device_id_type=pl.DeviceIdType.MESH,
            )

        drains = []
        ag_waves = []
        for w in range(N_WAVES):
            part_buf[w, 0] = partial_half(diag, w).astype(jnp.bfloat16)
            rs_diag = rs_send(diag, w, 2)
            rs_diag.start()
            part_buf[w, 1] = partial_half(right, w).astype(jnp.bfloat16)
            rs_right = rs_send(right, w, 0)
            rs_right.start()
            part_buf[w, 2] = partial_half(left, w).astype(jnp.bfloat16)
            rs_left = rs_send(left, w, 1)
            rs_left.start()
            own_buf[w] = partial_half(my, w)

            rs_right.wait_recv()
            rs_left.wait_recv()
            rs_diag.wait_recv()
            red = (own_buf[w] + rs_buf[w, 0].astype(jnp.float32)) + (
                rs_buf[w, 1].astype(jnp.float32)
                + rs_buf[w, 2].astype(jnp.float32))
            red_buf[w] = red.astype(jnp.bfloat16)

            ag_diag = ag_send(diag, w, 2)
            ag_diag.start()
            ag_right = ag_send(right, w, 0)
            ag_right.start()
            ag_left = ag_send(left, w, 1)
            ag_left.start()

            out_ref[pl.ds(my * CHUNK + w * HALF, HALF), :] = red

            ag_waves.append((ag_right, ag_left, ag_diag))
            drains += [rs_right, rs_left, rs_diag, ag_right, ag_left, ag_diag]

        for w, (ag_right, ag_left, ag_diag) in enumerate(ag_waves):
            ag_right.wait_recv()
            out_ref[pl.ds(left * CHUNK + w * HALF, HALF), :] = (
                ag_buf[w, 0].astype(jnp.float32))
            ag_left.wait_recv()
            out_ref[pl.ds(right * CHUNK + w * HALF, HALF), :] = (
                ag_buf[w, 1].astype(jnp.float32))
            ag_diag.wait_recv()
            out_ref[pl.ds(diag * CHUNK + w * HALF, HALF), :] = (
                ag_buf[w, 2].astype(jnp.float32))

        for d in drains:
            d.wait_send()

    return pl.pallas_call(
        body,
        out_shape=jax.ShapeDtypeStruct((M, D), jnp.float32),
        in_specs=[
            pl.BlockSpec(memory_space=pltpu.VMEM),
            pl.BlockSpec(memory_space=pltpu.VMEM),
            pl.BlockSpec(memory_space=pltpu.VMEM),
            pl.BlockSpec(memory_space=pltpu.VMEM),
        ],
        out_specs=pl.BlockSpec(memory_space=pltpu.VMEM),
        scratch_shapes=[
            pltpu.VMEM((M, 1024), jnp.bfloat16),
            pltpu.VMEM((1024, 2048), jnp.bfloat16),
            pltpu.VMEM((1024, 2048), jnp.bfloat16),
            pltpu.VMEM((2048, 1024), jnp.bfloat16),
            pltpu.VMEM((N_WAVES, 3, HALF, D), jnp.bfloat16),
            pltpu.VMEM((N_WAVES, HALF, D), jnp.float32),
            pltpu.VMEM((N_WAVES, HALF, D), jnp.bfloat16),
            pltpu.VMEM((N_WAVES, 3, HALF, D), jnp.bfloat16),
            pltpu.VMEM((N_WAVES, 3, HALF, D), jnp.bfloat16),
            pltpu.SemaphoreType.DMA((N_WAVES * 6,)),
            pltpu.SemaphoreType.DMA((N_WAVES * 3,)),
            pltpu.SemaphoreType.DMA((N_WAVES * 3,)),
        ],
        compiler_params=pltpu.CompilerParams(
            collective_id=0,
            vmem_limit_bytes=128 * 1024 * 1024,
        ),
    )(x, Wg, Wu, Wd)
